# Initial kernel scaffold; baseline (speedup 1.0000x reference)
import jax
import jax.numpy as jnp
from jax.experimental import pallas as pl


def kernel(h, edge_index, norm, weight, bias):
    raise NotImplementedError("write your pallas kernel here")


# speedup vs baseline: 7.5638x; 7.5638x over previous
import functools

import jax
import jax.numpy as jnp
from jax import lax
from jax.experimental import pallas as pl
from jax.experimental.pallas import tpu as pltpu
from jax.experimental.pallas import tpu_sc as plsc

N_NODES = 10000
N_EDGES = 320000
F = 128

NC = 2
NS = 16
NW = NC * NS
EPW = N_EDGES // NW
K = 100
NCHUNK = EPW // K
N_PAD = 10240
ROWS_PER_TILE = N_PAD // NS


def _linear_body(h_ref, norm_ref, w_ref, o_ref):
    o_ref[...] = norm_ref[...] * jnp.dot(
        h_ref[...], w_ref[...], preferred_element_type=jnp.float32)


def _tc_linear(h, norm, weight):
    blk = 1000
    grid = (N_NODES // blk,)
    return pl.pallas_call(
        _linear_body,
        grid=grid,
        in_specs=[
            pl.BlockSpec((blk, F), lambda i: (i, 0)),
            pl.BlockSpec((blk, 1), lambda i: (i, 0)),
            pl.BlockSpec((F, F), lambda i: (0, 0)),
        ],
        out_specs=pl.BlockSpec((blk, F), lambda i: (i, 0)),
        out_shape=jax.ShapeDtypeStruct((N_NODES, F), jnp.float32),
    )(h, norm, weight)


def _sc_body(x_hbm, src_hbm, dst_hbm, zeros_hbm, out_hbm,
             src_v, dst_v, rows_v, acc_sh, sem):
    c = lax.axis_index("c")
    s = lax.axis_index("s")

    pltpu.sync_copy(zeros_hbm, acc_sh.at[pl.ds(s * ROWS_PER_TILE, ROWS_PER_TILE)])

    pltpu.sync_copy(src_hbm.at[c, s], src_v)
    pltpu.sync_copy(dst_hbm.at[c, s], dst_v)
    plsc.subcore_barrier()

    def chunk(i, carry):
        pltpu.async_copy(x_hbm.at[src_v.at[i]], rows_v, sem).wait()
        pltpu.sync_copy(rows_v, acc_sh.at[dst_v.at[i]], add=True)
        return carry

    lax.fori_loop(0, NCHUNK, chunk, 0)
    plsc.subcore_barrier()

    pltpu.sync_copy(acc_sh.at[pl.ds(s * ROWS_PER_TILE, ROWS_PER_TILE)],
                    out_hbm.at[c, pl.ds(s * ROWS_PER_TILE, ROWS_PER_TILE)])


def _sc_aggregate(x, src_r, dst_r, zeros):
    mesh = plsc.VectorSubcoreMesh(
        core_axis_name="c", subcore_axis_name="s", num_cores=NC, num_subcores=NS)
    f = functools.partial(
        pl.kernel,
        out_type=jax.ShapeDtypeStruct((NC, N_PAD, F), jnp.float32),
        mesh=mesh,
        scratch_types=[
            pltpu.VMEM((NCHUNK, K), jnp.int32),
            pltpu.VMEM((NCHUNK, K), jnp.int32),
            pltpu.VMEM((K, F), jnp.float32),
            pltpu.VMEM_SHARED((N_PAD, F), jnp.float32),
            pltpu.SemaphoreType.DMA,
        ],
    )(_sc_body)
    return f(x, src_r, dst_r, zeros)


def _epilogue_body(p0_ref, p1_ref, norm_ref, b_ref, o_ref):
    agg = p0_ref[...] + p1_ref[...]
    o_ref[...] = jnp.maximum(agg * norm_ref[...] + b_ref[...], 0.0)


def _tc_epilogue(p0, p1, norm, bias2d):
    blk = 1000
    grid = (N_NODES // blk,)
    return pl.pallas_call(
        _epilogue_body,
        grid=grid,
        in_specs=[
            pl.BlockSpec((blk, F), lambda i: (i, 0)),
            pl.BlockSpec((blk, F), lambda i: (i, 0)),
            pl.BlockSpec((blk, 1), lambda i: (i, 0)),
            pl.BlockSpec((1, F), lambda i: (0, 0)),
        ],
        out_specs=pl.BlockSpec((blk, F), lambda i: (i, 0)),
        out_shape=jax.ShapeDtypeStruct((N_NODES, F), jnp.float32),
    )(p0, p1, norm, bias2d)


def kernel(h, edge_index, norm, weight, bias):
    x = _tc_linear(h, norm, weight)
    src_r = edge_index[0].reshape(NC, NS, NCHUNK, K)
    dst_r = edge_index[1].reshape(NC, NS, NCHUNK, K)
    zeros = jnp.zeros((ROWS_PER_TILE, F), jnp.float32)
    partials = _sc_aggregate(x, src_r, dst_r, zeros)
    return _tc_epilogue(partials[0, :N_NODES], partials[1, :N_NODES], norm, bias.reshape(1, F))


# speedup vs baseline: 10.9305x; 1.4451x over previous
import functools

import jax
import jax.numpy as jnp
from jax import lax
from jax.experimental import pallas as pl
from jax.experimental.pallas import tpu as pltpu
from jax.experimental.pallas import tpu_sc as plsc

N_NODES = 10000
N_EDGES = 320000
F = 128

NC = 2
NS = 16
NW = NC * NS
EPW = N_EDGES // NW
K = 100
NCHUNK = EPW // K
N_PAD = 10240
ROWS_PER_TILE = N_PAD // NS


def _linear_body(h_ref, norm_ref, w_ref, o_ref):
    o_ref[...] = norm_ref[...] * jnp.dot(
        h_ref[...], w_ref[...], preferred_element_type=jnp.float32)


def _tc_linear(h, norm, weight):
    blk = 1000
    grid = (N_NODES // blk,)
    return pl.pallas_call(
        _linear_body,
        grid=grid,
        in_specs=[
            pl.BlockSpec((blk, F), lambda i: (i, 0)),
            pl.BlockSpec((blk, 1), lambda i: (i, 0)),
            pl.BlockSpec((F, F), lambda i: (0, 0)),
        ],
        out_specs=pl.BlockSpec((blk, F), lambda i: (i, 0)),
        out_shape=jax.ShapeDtypeStruct((N_NODES, F), jnp.float32),
    )(h, norm, weight)


def _sc_body(x_hbm, src_hbm, dst_hbm, zeros_hbm, out_hbm,
             src_v, didx, rows_v0, rows_v1, acc_sh, gsem0, gsem1, dsem0, dsem1):
    c = lax.axis_index("c")
    s = lax.axis_index("s")

    pltpu.sync_copy(zeros_hbm, acc_sh.at[pl.ds(s * ROWS_PER_TILE, ROWS_PER_TILE)])

    pltpu.sync_copy(src_hbm.at[c, s], src_v)
    plsc.subcore_barrier()

    pltpu.async_copy(dst_hbm.at[c, s, 0], didx.at[0], dsem0)
    pltpu.async_copy(x_hbm.at[src_v.at[0]], rows_v0, gsem0)
    pltpu.async_copy(dst_hbm.at[c, s, 1], didx.at[1], dsem1)
    pltpu.async_copy(x_hbm.at[src_v.at[1]], rows_v1, gsem1)

    def pair(j, carry):
        i = 2 * j
        pltpu.make_async_copy(x_hbm.at[src_v.at[i]], rows_v0, gsem0).wait()
        pltpu.make_async_copy(dst_hbm.at[c, s, i], didx.at[0], dsem0).wait()
        pltpu.sync_copy(rows_v0, acc_sh.at[didx.at[0]], add=True)

        @pl.when(i + 2 < NCHUNK)
        def _():
            pltpu.async_copy(dst_hbm.at[c, s, i + 2], didx.at[0], dsem0)
            pltpu.async_copy(x_hbm.at[src_v.at[i + 2]], rows_v0, gsem0)

        pltpu.make_async_copy(x_hbm.at[src_v.at[i + 1]], rows_v1, gsem1).wait()
        pltpu.make_async_copy(dst_hbm.at[c, s, i + 1], didx.at[1], dsem1).wait()
        pltpu.sync_copy(rows_v1, acc_sh.at[didx.at[1]], add=True)

        @pl.when(i + 3 < NCHUNK)
        def _():
            pltpu.async_copy(dst_hbm.at[c, s, i + 3], didx.at[1], dsem1)
            pltpu.async_copy(x_hbm.at[src_v.at[i + 3]], rows_v1, gsem1)
        return carry

    lax.fori_loop(0, NCHUNK // 2, pair, 0)
    plsc.subcore_barrier()

    pltpu.sync_copy(acc_sh.at[pl.ds(s * ROWS_PER_TILE, ROWS_PER_TILE)],
                    out_hbm.at[c, pl.ds(s * ROWS_PER_TILE, ROWS_PER_TILE)])


def _sc_aggregate(x, src_r, dst_r, zeros):
    mesh = plsc.VectorSubcoreMesh(
        core_axis_name="c", subcore_axis_name="s", num_cores=NC, num_subcores=NS)
    f = functools.partial(
        pl.kernel,
        out_type=jax.ShapeDtypeStruct((NC, N_PAD, F), jnp.float32),
        mesh=mesh,
        scratch_types=[
            pltpu.VMEM((NCHUNK, K), jnp.int32),
            pltpu.VMEM((2, K), jnp.int32),
            pltpu.VMEM((K, F), jnp.float32),
            pltpu.VMEM((K, F), jnp.float32),
            pltpu.VMEM_SHARED((N_PAD, F), jnp.float32),
            pltpu.SemaphoreType.DMA,
            pltpu.SemaphoreType.DMA,
            pltpu.SemaphoreType.DMA,
            pltpu.SemaphoreType.DMA,
        ],
    )(_sc_body)
    return f(x, src_r, dst_r, zeros)


def _epilogue_body(p0_ref, p1_ref, norm_ref, b_ref, o_ref):
    agg = p0_ref[...] + p1_ref[...]
    o_ref[...] = jnp.maximum(agg * norm_ref[...] + b_ref[...], 0.0)


def _tc_epilogue(p0, p1, norm, bias2d):
    blk = 1000
    grid = (N_NODES // blk,)
    return pl.pallas_call(
        _epilogue_body,
        grid=grid,
        in_specs=[
            pl.BlockSpec((blk, F), lambda i: (i, 0)),
            pl.BlockSpec((blk, F), lambda i: (i, 0)),
            pl.BlockSpec((blk, 1), lambda i: (i, 0)),
            pl.BlockSpec((1, F), lambda i: (0, 0)),
        ],
        out_specs=pl.BlockSpec((blk, F), lambda i: (i, 0)),
        out_shape=jax.ShapeDtypeStruct((N_NODES, F), jnp.float32),
    )(p0, p1, norm, bias2d)


def kernel(h, edge_index, norm, weight, bias):
    x = _tc_linear(h, norm, weight)
    src_r = edge_index[0].reshape(NC, NS, NCHUNK, K)
    dst_r = edge_index[1].reshape(NC, NS, NCHUNK, K)
    zeros = jnp.zeros((ROWS_PER_TILE, F), jnp.float32)
    partials = _sc_aggregate(x, src_r, dst_r, zeros)
    return _tc_epilogue(partials[0, :N_NODES], partials[1, :N_NODES], norm, bias.reshape(1, F))


# speedup vs baseline: 12.1258x; 1.1094x over previous
import functools

import jax
import jax.numpy as jnp
from jax import lax
from jax.experimental import pallas as pl
from jax.experimental.pallas import tpu as pltpu
from jax.experimental.pallas import tpu_sc as plsc

N_NODES = 10000
N_EDGES = 320000
F = 128

NC = 2
NS = 16
NW = NC * NS
EPW = N_EDGES // NW
K = 100
NCHUNK = EPW // K
N_PAD = 10240
ROWS_PER_TILE = N_PAD // NS


def _linear_body(h_ref, norm_ref, w_ref, o_ref):
    o_ref[...] = norm_ref[...] * jnp.dot(
        h_ref[...], w_ref[...], preferred_element_type=jnp.float32)


def _tc_linear(h, norm, weight):
    blk = 1000
    grid = (N_NODES // blk,)
    return pl.pallas_call(
        _linear_body,
        grid=grid,
        in_specs=[
            pl.BlockSpec((blk, F), lambda i: (i, 0)),
            pl.BlockSpec((blk, 1), lambda i: (i, 0)),
            pl.BlockSpec((F, F), lambda i: (0, 0)),
        ],
        out_specs=pl.BlockSpec((blk, F), lambda i: (i, 0)),
        out_shape=jax.ShapeDtypeStruct((N_NODES, F), jnp.float32),
    )(h, norm, weight)


def _sc_body(x_hbm, edges_hbm, zeros_hbm, out_hbm,
             src_v, didx, rows_v0, rows_v1, acc_sh, gsem0, gsem1, dsem0, dsem1):
    c = lax.axis_index("c")
    s = lax.axis_index("s")

    pltpu.sync_copy(zeros_hbm, acc_sh.at[pl.ds(s * ROWS_PER_TILE, ROWS_PER_TILE)])

    pltpu.sync_copy(edges_hbm.at[0, c, s], src_v)
    plsc.subcore_barrier()

    pltpu.async_copy(edges_hbm.at[1, c, s, 0], didx.at[0], dsem0)
    pltpu.async_copy(x_hbm.at[src_v.at[0]], rows_v0, gsem0)
    pltpu.async_copy(edges_hbm.at[1, c, s, 1], didx.at[1], dsem1)
    pltpu.async_copy(x_hbm.at[src_v.at[1]], rows_v1, gsem1)

    def pair(j, carry):
        i = 2 * j
        pltpu.make_async_copy(x_hbm.at[src_v.at[i]], rows_v0, gsem0).wait()
        pltpu.make_async_copy(edges_hbm.at[1, c, s, i], didx.at[0], dsem0).wait()
        pltpu.sync_copy(rows_v0, acc_sh.at[didx.at[0]], add=True)

        @pl.when(i + 2 < NCHUNK)
        def _():
            pltpu.async_copy(edges_hbm.at[1, c, s, i + 2], didx.at[0], dsem0)
            pltpu.async_copy(x_hbm.at[src_v.at[i + 2]], rows_v0, gsem0)

        pltpu.make_async_copy(x_hbm.at[src_v.at[i + 1]], rows_v1, gsem1).wait()
        pltpu.make_async_copy(edges_hbm.at[1, c, s, i + 1], didx.at[1], dsem1).wait()
        pltpu.sync_copy(rows_v1, acc_sh.at[didx.at[1]], add=True)

        @pl.when(i + 3 < NCHUNK)
        def _():
            pltpu.async_copy(edges_hbm.at[1, c, s, i + 3], didx.at[1], dsem1)
            pltpu.async_copy(x_hbm.at[src_v.at[i + 3]], rows_v1, gsem1)
        return carry

    lax.fori_loop(0, NCHUNK // 2, pair, 0)
    plsc.subcore_barrier()

    pltpu.sync_copy(acc_sh.at[pl.ds(s * ROWS_PER_TILE, ROWS_PER_TILE)],
                    out_hbm.at[c, pl.ds(s * ROWS_PER_TILE, ROWS_PER_TILE)])


def _sc_aggregate(x, edges, zeros):
    mesh = plsc.VectorSubcoreMesh(
        core_axis_name="c", subcore_axis_name="s", num_cores=NC, num_subcores=NS)
    f = functools.partial(
        pl.kernel,
        out_type=jax.ShapeDtypeStruct((NC, N_PAD, F), jnp.float32),
        mesh=mesh,
        scratch_types=[
            pltpu.VMEM((NCHUNK, K), jnp.int32),
            pltpu.VMEM((2, K), jnp.int32),
            pltpu.VMEM((K, F), jnp.float32),
            pltpu.VMEM((K, F), jnp.float32),
            pltpu.VMEM_SHARED((N_PAD, F), jnp.float32),
            pltpu.SemaphoreType.DMA,
            pltpu.SemaphoreType.DMA,
            pltpu.SemaphoreType.DMA,
            pltpu.SemaphoreType.DMA,
        ],
    )(_sc_body)
    return f(x, edges, zeros)


def _epilogue_body(p_ref, norm_ref, b_ref, o_ref):
    agg = p_ref[0] + p_ref[1]
    o_ref[...] = jnp.maximum(agg * norm_ref[...] + b_ref[...], 0.0)


def _tc_epilogue(partials, norm, bias2d):
    blk = 1000
    grid = (N_NODES // blk,)
    return pl.pallas_call(
        _epilogue_body,
        grid=grid,
        in_specs=[
            pl.BlockSpec((NC, blk, F), lambda i: (0, i, 0)),
            pl.BlockSpec((blk, 1), lambda i: (i, 0)),
            pl.BlockSpec((1, F), lambda i: (0, 0)),
        ],
        out_specs=pl.BlockSpec((blk, F), lambda i: (i, 0)),
        out_shape=jax.ShapeDtypeStruct((N_NODES, F), jnp.float32),
    )(partials, norm, bias2d)


def kernel(h, edge_index, norm, weight, bias):
    x = _tc_linear(h, norm, weight)
    edges = edge_index.reshape(2, NC, NS, NCHUNK, K)
    zeros = jnp.zeros((ROWS_PER_TILE, F), jnp.float32)
    partials = _sc_aggregate(x, edges, zeros)
    return _tc_epilogue(partials, norm, bias.reshape(1, F))


# speedup vs baseline: 12.6772x; 1.0455x over previous
import functools

import jax
import jax.numpy as jnp
from jax import lax
from jax.experimental import pallas as pl
from jax.experimental.pallas import tpu as pltpu
from jax.experimental.pallas import tpu_sc as plsc

N_NODES = 10000
N_EDGES = 320000
F = 128

NC = 2
NS = 16
NW = NC * NS
EPW = N_EDGES // NW
K = 100
NCHUNK = EPW // K
N_PAD = 10240
ROWS_PER_TILE = N_PAD // NS
ZCOPY = ROWS_PER_TILE // K


def _linear_body(h_ref, norm_ref, w_ref, o_ref):
    o_ref[...] = norm_ref[...] * jnp.dot(
        h_ref[...], w_ref[...], preferred_element_type=jnp.float32)


def _tc_linear(h, norm, weight):
    blk = 2000
    grid = (N_NODES // blk,)
    return pl.pallas_call(
        _linear_body,
        grid=grid,
        in_specs=[
            pl.BlockSpec((blk, F), lambda i: (i, 0)),
            pl.BlockSpec((blk, 1), lambda i: (i, 0)),
            pl.BlockSpec((F, F), lambda i: (0, 0)),
        ],
        out_specs=pl.BlockSpec((blk, F), lambda i: (i, 0)),
        out_shape=jax.ShapeDtypeStruct((N_NODES, F), jnp.float32),
    )(h, norm, weight)


def _sc_body(x_hbm, edges_hbm, out_hbm,
             src_v, didx, rows_v0, rows_v1, acc_sh, gsem0, gsem1, dsem0, dsem1):
    c = lax.axis_index("c")
    s = lax.axis_index("s")

    zv = jnp.zeros((16,), jnp.float32)

    def zrow(r, carry):
        for l in range(F // 16):
            rows_v0[r, pl.ds(l * 16, 16)] = zv
        return carry

    lax.fori_loop(0, K, zrow, 0)
    for z in range(ZCOPY):
        pltpu.sync_copy(rows_v0, acc_sh.at[pl.ds(s * ROWS_PER_TILE + z * K, K)])

    pltpu.sync_copy(edges_hbm.at[0, c, s], src_v)
    plsc.subcore_barrier()

    pltpu.async_copy(edges_hbm.at[1, c, s, 0], didx.at[0], dsem0)
    pltpu.async_copy(x_hbm.at[src_v.at[0]], rows_v0, gsem0)
    pltpu.async_copy(edges_hbm.at[1, c, s, 1], didx.at[1], dsem1)
    pltpu.async_copy(x_hbm.at[src_v.at[1]], rows_v1, gsem1)

    def chunk(i, rows_v, gsem, didx_row, dsem):
        pltpu.make_async_copy(x_hbm.at[src_v.at[0]], rows_v, gsem).wait()
        pltpu.make_async_copy(edges_hbm.at[1, c, s, 0], didx_row, dsem).wait()
        pltpu.sync_copy(rows_v, acc_sh.at[didx_row], add=True)

        @pl.when(i + 2 < NCHUNK)
        def _():
            pltpu.async_copy(edges_hbm.at[1, c, s, i + 2], didx_row, dsem)
            pltpu.async_copy(x_hbm.at[src_v.at[i + 2]], rows_v, gsem)

    def pair(j, carry):
        i = 2 * j
        chunk(i, rows_v0, gsem0, didx.at[0], dsem0)
        chunk(i + 1, rows_v1, gsem1, didx.at[1], dsem1)
        return carry

    lax.fori_loop(0, NCHUNK // 2, pair, 0)
    if NCHUNK % 2:
        chunk(NCHUNK - 1, rows_v0, gsem0, didx.at[0], dsem0)
    plsc.subcore_barrier()

    pltpu.sync_copy(acc_sh.at[pl.ds(s * ROWS_PER_TILE, ROWS_PER_TILE)],
                    out_hbm.at[c, pl.ds(s * ROWS_PER_TILE, ROWS_PER_TILE)])


def _sc_aggregate(x, edges):
    mesh = plsc.VectorSubcoreMesh(
        core_axis_name="c", subcore_axis_name="s", num_cores=NC, num_subcores=NS)
    f = functools.partial(
        pl.kernel,
        out_type=jax.ShapeDtypeStruct((NC, N_PAD, F), jnp.float32),
        mesh=mesh,
        scratch_types=[
            pltpu.VMEM((NCHUNK, K), jnp.int32),
            pltpu.VMEM((2, K), jnp.int32),
            pltpu.VMEM((K, F), jnp.float32),
            pltpu.VMEM((K, F), jnp.float32),
            pltpu.VMEM_SHARED((N_PAD, F), jnp.float32),
            pltpu.SemaphoreType.DMA,
            pltpu.SemaphoreType.DMA,
            pltpu.SemaphoreType.DMA,
            pltpu.SemaphoreType.DMA,
        ],
    )(_sc_body)
    return f(x, edges)


def _epilogue_body(p_ref, norm_ref, b_ref, o_ref):
    agg = p_ref[0] + p_ref[1]
    o_ref[...] = jnp.maximum(agg * norm_ref[...] + b_ref[...], 0.0)


def _tc_epilogue(partials, norm, bias2d):
    blk = 1000
    grid = (N_NODES // blk,)
    return pl.pallas_call(
        _epilogue_body,
        grid=grid,
        in_specs=[
            pl.BlockSpec((NC, blk, F), lambda i: (0, i, 0)),
            pl.BlockSpec((blk, 1), lambda i: (i, 0)),
            pl.BlockSpec((1, F), lambda i: (0, 0)),
        ],
        out_specs=pl.BlockSpec((blk, F), lambda i: (i, 0)),
        out_shape=jax.ShapeDtypeStruct((N_NODES, F), jnp.float32),
    )(partials, norm, bias2d)


def kernel(h, edge_index, norm, weight, bias):
    x = _tc_linear(h, norm, weight)
    edges = edge_index.reshape(2, NC, NS, NCHUNK, K)
    partials = _sc_aggregate(x, edges)
    return _tc_epilogue(partials, norm, bias.reshape(1, F))
